# baseline (speedup 1.0000x reference)
import jax
import jax.numpy as jnp
from jax import lax
from jax.experimental import pallas as pl
from jax.experimental.pallas import tpu as pltpu

_UPSAMPLE = 32
_VMEM_LIMIT = 56 * 1024 * 1024


def _band_rel(W, cin, cout):
    r = jnp.arange(W * cin)[:, None] // cin
    c = jnp.arange(W * cout)[None, :] // cout
    return r - c + 1


def _band2d(w, W):
    _, _, cin, cout = w.shape
    rel = _band_rel(W, cin, cout)
    band = jnp.zeros((3, W * cin, W * cout), w.dtype)
    for kw in range(3):
        m = (rel == kw).astype(w.dtype)
        band = band + m * jnp.tile(w[:, kw], (1, W, W))
    return band


def _band3d(w, W):
    _, _, _, cin, cout = w.shape
    rel = _band_rel(W, cin, cout)
    band = jnp.zeros((3, 3, W * cin, W * cout), w.dtype)
    for kw in range(3):
        m = (rel == kw).astype(w.dtype)
        band = band + m * jnp.tile(w[:, :, kw], (1, 1, W, W))
    return band


def _pool_cols(W, cout):
    r = jnp.arange(W * cout)[:, None]
    c = jnp.arange((W // 2) * cout)[None, :]
    ok = ((r // cout) // 2 == c // cout) & (r % cout == c % cout)
    return jnp.where(ok, 0.25, 0.0).astype(jnp.float32)


def _conv_bn_relu(x, w_ref, s_ref, b_ref, Nb, H, K, Nout):
    z = jnp.zeros((Nb, 1, K), jnp.float32)
    xp = jnp.concatenate([z, x, z], axis=1).astype(jnp.bfloat16)
    acc = jnp.zeros((Nb * H, Nout), jnp.float32)
    for kh in range(3):
        acc += jnp.dot(xp[:, kh:kh + H, :].reshape(Nb * H, K), w_ref[kh],
                       preferred_element_type=jnp.float32)
    y = jnp.maximum(acc * s_ref[...] + b_ref[...], 0.0)
    return y.reshape(Nb, H, Nout)


def _rowpool_sum(x, Nb, H, C):
    Hp = H // 2
    pr = (lax.broadcasted_iota(jnp.int32, (Hp, H), 1) // 2 ==
          lax.broadcasted_iota(jnp.int32, (Hp, H), 0)).astype(jnp.float32)
    return jnp.concatenate(
        [jnp.dot(pr, x[n], preferred_element_type=jnp.float32)[None]
         for n in range(Nb)], axis=0)


def _conv3d_even_t(x, w_ref, b_ref, Nb, T, H, K, Nout):
    To = T // 2
    xr = x.reshape(Nb, To, 2, H, K)
    zt = jnp.zeros((Nb, 1, H, K), jnp.float32)
    xe = jnp.concatenate([zt, xr[:, :, 1]], axis=1)
    xo = jnp.concatenate([xr[:, :, 0], zt], axis=1)
    zh = jnp.zeros((Nb, To + 1, 1, K), jnp.float32)
    xe = jnp.concatenate([zh, xe, zh], axis=2).astype(jnp.bfloat16)
    xo = jnp.concatenate([zh, xo, zh], axis=2).astype(jnp.bfloat16)
    acc = jnp.zeros((Nb * To * H, Nout), jnp.float32)
    for kt, (src, a) in enumerate(((xe, 0), (xo, 0), (xe, 1))):
        for kh in range(3):
            sl = src[:, a:a + To, kh:kh + H, :].reshape(Nb * To * H, K)
            acc += jnp.dot(sl, w_ref[kt, kh],
                           preferred_element_type=jnp.float32)
    return (acc + b_ref[...]).reshape(Nb, To, H, Nout)


def _maxpool_hw(x, H, W, C, p):
    Hout = (H + 2 * p - 2) // 2 + 1
    Wout = (W + 2 * p - 2) // 2 + 1
    rows = []
    for ho in range(Hout):
        r = None
        for i in (2 * ho - p, 2 * ho - p + 1):
            if 0 <= i < H:
                v = x[:, :, i, :]
                r = v if r is None else jnp.maximum(r, v)
        rows.append(r[:, :, None, :])
    x = jnp.concatenate(rows, axis=2)
    cols = []
    for wo in range(Wout):
        c = None
        for j in (2 * wo - p, 2 * wo - p + 1):
            if 0 <= j < W:
                v = x[..., j * C:(j + 1) * C]
                c = v if c is None else jnp.maximum(c, v)
        cols.append(c)
    return jnp.concatenate(cols, axis=-1)


def _make_audio_body(Nb, dims):
    def body(*refs):
        x = refs[0][...].astype(jnp.float32)
        o_ref = refs[-1]
        for i, (H, Kin, Nout) in enumerate(dims):
            w1, w2, s1, b1, s2, b2 = refs[1 + 6 * i: 7 + 6 * i]
            x = _conv_bn_relu(x, w1, s1, b1, Nb, H, Kin, Nout)
            x = _conv_bn_relu(x, w2, s2, b2, Nb, H, Nout, Nout)
            if i < len(dims) - 1:
                x = _rowpool_sum(x, Nb, H, Nout)
        o_ref[...] = x.astype(o_ref.dtype)
    return body


def _make_video_body(Nb, dims, C_last, nclass):
    nL = len(dims)

    def body(*refs):
        x = refs[0][...].astype(jnp.float32)
        a_ref, fw_ref, fb_ref = refs[1 + 2 * nL], refs[2 + 2 * nL], refs[3 + 2 * nL]
        tfv_ref, frame_ref = refs[-2], refs[-1]
        for i, (T, H, W, Cin, Cout, p) in enumerate(dims):
            w_ref, b_ref = refs[1 + 2 * i], refs[2 + 2 * i]
            x = _conv3d_even_t(x, w_ref, b_ref, Nb, T, H, W * Cin, W * Cout)
            x = _maxpool_hw(x, H, W, Cout, p)
        Tf, Hf = x.shape[1], x.shape[2]
        Wf = x.shape[3] // C_last
        acc = x[:, :, 0, :]
        for h in range(1, Hf):
            acc = acc + x[:, :, h, :]
        m = acc[..., :C_last]
        for w in range(1, Wf):
            m = m + acc[..., w * C_last:(w + 1) * C_last]
        tfv = m * (1.0 / float(Hf * Wf))
        tfv_ref[...] = tfv.astype(tfv_ref.dtype)
        a = a_ref[...].astype(jnp.float32)
        fused = jnp.maximum(jnp.max(a, axis=1), jnp.max(tfv, axis=1))
        z = jnp.dot(fused, fw_ref[...],
                    preferred_element_type=jnp.float32) + fb_ref[...]
        frame = 1.0 / (1.0 + jnp.exp(-z))
        frame_ref[...] = frame[:, None, :].astype(frame_ref.dtype)
    return body


def kernel(feature, video_feature, a0_w1, a0_w2, a0_s1, a0_b1, a0_s2, a0_b2,
           a1_w1, a1_w2, a1_s1, a1_b1, a1_s2, a1_b2,
           a2_w1, a2_w2, a2_s1, a2_b1, a2_s2, a2_b2,
           a3_w1, a3_w2, a3_s1, a3_b1, a3_s2, a3_b2,
           a4_w1, a4_w2, a4_s1, a4_b1, a4_s2, a4_b2,
           a5_w1, a5_w2, a5_s1, a5_b1, a5_s2, a5_b2,
           v0_w, v0_b, v1_w, v1_b, v2_w, v2_b,
           v3_w, v3_b, v4_w, v4_b, v5_w, v5_b, fc_w, fc_b):
    ablocks = [(a0_w1, a0_w2, a0_s1, a0_b1, a0_s2, a0_b2),
               (a1_w1, a1_w2, a1_s1, a1_b1, a1_s2, a1_b2),
               (a2_w1, a2_w2, a2_s1, a2_b1, a2_s2, a2_b2),
               (a3_w1, a3_w2, a3_s1, a3_b1, a3_s2, a3_b2),
               (a4_w1, a4_w2, a4_s1, a4_b1, a4_s2, a4_b2),
               (a5_w1, a5_w2, a5_s1, a5_b1, a5_s2, a5_b2)]
    vconvs = [(v0_w, v0_b), (v1_w, v1_b), (v2_w, v2_b),
              (v3_w, v3_b), (v4_w, v4_b), (v5_w, v5_b)]

    N, Ta, Fa = feature.shape
    _, Cv, Tv, Hv, Wv = video_feature.shape
    nclass = fc_w.shape[0]

    Nba = 8 if N % 8 == 0 else 1
    adims, ops, specs = [], [feature], [
        pl.BlockSpec((Nba, Ta, Fa), lambda n: (n, 0, 0))]
    H, W, Cin = Ta, Fa, 1
    pc_prev = None
    for i, (w1, w2, s1, b1, s2, b2) in enumerate(ablocks):
        Cout = w1.shape[3]
        w1b = _band2d(w1, W)
        if pc_prev is not None:
            w1b = jnp.matmul(pc_prev, w1b)
        adims.append((H, w1b.shape[1], W * Cout))
        ops += [w1b.astype(jnp.bfloat16),
                _band2d(w2, W).astype(jnp.bfloat16),
                jnp.tile(s1, W)[None, :], jnp.tile(b1, W)[None, :],
                jnp.tile(s2, W)[None, :], jnp.tile(b2, W)[None, :]]
        specs += [pl.BlockSpec(ops[-6].shape, lambda n: (0, 0, 0)),
                  pl.BlockSpec(ops[-5].shape, lambda n: (0, 0, 0)),
                  pl.BlockSpec(ops[-4].shape, lambda n: (0, 0)),
                  pl.BlockSpec(ops[-3].shape, lambda n: (0, 0)),
                  pl.BlockSpec(ops[-2].shape, lambda n: (0, 0)),
                  pl.BlockSpec(ops[-1].shape, lambda n: (0, 0))]
        if i < len(ablocks) - 1:
            pc_prev = _pool_cols(W, Cout)
            H, W = H // 2, W // 2
        Cin = Cout
    C = Cin
    tf_a = pl.pallas_call(
        _make_audio_body(Nba, adims),
        out_shape=jax.ShapeDtypeStruct((N, H, C), feature.dtype),
        grid_spec=pltpu.PrefetchScalarGridSpec(
            num_scalar_prefetch=0, grid=(N // Nba,), in_specs=specs,
            out_specs=pl.BlockSpec((Nba, H, C), lambda n: (n, 0, 0))),
        compiler_params=pltpu.CompilerParams(
            dimension_semantics=("parallel",),
            vmem_limit_bytes=_VMEM_LIMIT),
    )(*ops)

    Nbv = 4 if N % 4 == 0 else 1
    xv = jnp.transpose(video_feature, (0, 2, 3, 4, 1)).reshape(N, Tv, Hv,
                                                               Wv * Cv)
    vdims, vops = [], [xv]
    vspecs = [pl.BlockSpec((Nbv, Tv, Hv, Wv * Cv), lambda n: (n, 0, 0, 0))]
    pool_pads = (1, 1, 0, 1, 1, 1)
    T, Hh, Ww, Cin = Tv, Hv, Wv, Cv
    for p, (w, b) in zip(pool_pads, vconvs):
        Cout = w.shape[4]
        vdims.append((T, Hh, Ww, Cin, Cout, p))
        vops += [_band3d(w, Ww).astype(jnp.bfloat16), jnp.tile(b, Ww)[None, :]]
        vspecs += [pl.BlockSpec(vops[-2].shape, lambda n: (0, 0, 0, 0)),
                   pl.BlockSpec(vops[-1].shape, lambda n: (0, 0))]
        T = T // 2
        Hh = (Hh + 2 * p - 2) // 2 + 1
        Ww = (Ww + 2 * p - 2) // 2 + 1
        Cin = Cout
    vops += [tf_a, fc_w.T, fc_b[None, :]]
    vspecs += [pl.BlockSpec((Nbv,) + tf_a.shape[1:], lambda n: (n, 0, 0)),
               pl.BlockSpec(fc_w.T.shape, lambda n: (0, 0)),
               pl.BlockSpec((1, nclass), lambda n: (0, 0))]
    tf_v, frame3 = pl.pallas_call(
        _make_video_body(Nbv, vdims, C, nclass),
        out_shape=[jax.ShapeDtypeStruct((N, T, C), feature.dtype),
                   jax.ShapeDtypeStruct((N, 1, nclass), feature.dtype)],
        grid_spec=pltpu.PrefetchScalarGridSpec(
            num_scalar_prefetch=0, grid=(N // Nbv,), in_specs=vspecs,
            out_specs=[pl.BlockSpec((Nbv, T, C), lambda n: (n, 0, 0)),
                       pl.BlockSpec((Nbv, 1, nclass), lambda n: (n, 0, 0))]),
        compiler_params=pltpu.CompilerParams(
            dimension_semantics=("parallel",),
            vmem_limit_bytes=_VMEM_LIMIT),
    )(*vops)

    frame = frame3[:, 0, :]
    framewise = jnp.repeat(frame[:, None, :], _UPSAMPLE, axis=1)
    return {"framewise_output": framewise, "clipwise_output": frame,
            "tf_maps_a": tf_a, "tf_maps_v": tf_v}


# speedup vs baseline: 1.3838x; 1.3838x over previous
import jax
import jax.numpy as jnp
from jax import lax
from jax.experimental import pallas as pl
from jax.experimental.pallas import tpu as pltpu

_UPSAMPLE = 32
_VMEM_LIMIT = 56 * 1024 * 1024


def _band2d(w, W):
    _, _, cin, cout = w.shape
    rel = jnp.arange(W)[:, None] - jnp.arange(W)[None, :] + 1
    ok = (rel >= 0) & (rel <= 2)
    taps = w[:, jnp.clip(rel, 0, 2)] * ok[None, :, :, None, None].astype(w.dtype)
    return taps.transpose(0, 1, 3, 2, 4).reshape(3, W * cin, W * cout)


def _band3d(w, W):
    return jnp.stack([_band2d(w[kt], W) for kt in range(3)])


def _pool_cols(W, cout):
    r = jnp.arange(W * cout)[:, None]
    c = jnp.arange((W // 2) * cout)[None, :]
    ok = ((r // cout) // 2 == c // cout) & (r % cout == c % cout)
    return jnp.where(ok, 0.25, 0.0).astype(jnp.float32)


def _conv_bn_relu(x, w_ref, s_ref, b_ref, Nb, H, K, Nout):
    z = jnp.zeros((Nb, 1, K), jnp.float32)
    xp = jnp.concatenate([z, x, z], axis=1).astype(jnp.bfloat16)
    acc = jnp.zeros((Nb * H, Nout), jnp.float32)
    for kh in range(3):
        acc += jnp.dot(xp[:, kh:kh + H, :].reshape(Nb * H, K), w_ref[kh],
                       preferred_element_type=jnp.float32)
    y = jnp.maximum(acc * s_ref[...] + b_ref[...], 0.0)
    return y.reshape(Nb, H, Nout)


def _rowpool_sum(x, Nb, H, C):
    Hp = H // 2
    pr = (lax.broadcasted_iota(jnp.int32, (Hp, H), 1) // 2 ==
          lax.broadcasted_iota(jnp.int32, (Hp, H), 0)).astype(jnp.float32)
    return jnp.concatenate(
        [jnp.dot(pr, x[n], preferred_element_type=jnp.float32)[None]
         for n in range(Nb)], axis=0)


def _conv3d_taps(x, wsel, Nb, T, H, K, acc):
    To = T // 2
    xr = x.reshape(Nb, To, 2, H, K)
    zt = jnp.zeros((Nb, 1, H, K), jnp.float32)
    xe = jnp.concatenate([zt, xr[:, :, 1]], axis=1)
    xo = jnp.concatenate([xr[:, :, 0], zt], axis=1)
    zh = jnp.zeros((Nb, To + 1, 1, K), jnp.float32)
    xe = jnp.concatenate([zh, xe, zh], axis=2).astype(jnp.bfloat16)
    xo = jnp.concatenate([zh, xo, zh], axis=2).astype(jnp.bfloat16)
    for kt, (src, a) in enumerate(((xe, 0), (xo, 0), (xe, 1))):
        for kh in range(3):
            sl = src[:, a:a + To, kh:kh + H, :].reshape(Nb * To * H, K)
            acc += jnp.dot(sl, wsel(kt, kh),
                           preferred_element_type=jnp.float32)
    return acc


def _maxpool_hw(x, H, W, C, p):
    Hout = (H + 2 * p - 2) // 2 + 1
    Wout = (W + 2 * p - 2) // 2 + 1
    rows = []
    for ho in range(Hout):
        r = None
        for i in (2 * ho - p, 2 * ho - p + 1):
            if 0 <= i < H:
                v = x[:, :, i, :]
                r = v if r is None else jnp.maximum(r, v)
        rows.append(r[:, :, None, :])
    x = jnp.concatenate(rows, axis=2)
    cols = []
    for wo in range(Wout):
        c = None
        for j in (2 * wo - p, 2 * wo - p + 1):
            if 0 <= j < W:
                v = x[..., j * C:(j + 1) * C]
                c = v if c is None else jnp.maximum(c, v)
        cols.append(c)
    return jnp.concatenate(cols, axis=-1)


def _make_audio_body(Nb, dims):
    def body(*refs):
        x = refs[0][...].astype(jnp.float32)
        o_ref = refs[-1]
        for i, (H, Kin, Nout) in enumerate(dims):
            w1, w2, s1, b1, s2, b2 = refs[1 + 6 * i: 7 + 6 * i]
            x = _conv_bn_relu(x, w1, s1, b1, Nb, H, Kin, Nout)
            x = _conv_bn_relu(x, w2, s2, b2, Nb, H, Nout, Nout)
            if i < len(dims) - 1:
                x = _rowpool_sum(x, Nb, H, Nout)
        o_ref[...] = x.astype(o_ref.dtype)
    return body


def _make_video_body(Nb, dims, C_last, nclass):
    nL = len(dims)

    def body(*refs):
        a_ref, fw_ref, fb_ref = refs[1 + 2 * nL], refs[2 + 2 * nL], refs[3 + 2 * nL]
        tfv_ref, frame_ref = refs[-2], refs[-1]
        x = None
        for i, (T, H, W, Cin, Cout, p) in enumerate(dims):
            w_ref, b_ref = refs[1 + 2 * i], refs[2 + 2 * i]
            acc = jnp.zeros((Nb * (T // 2) * H, W * Cout), jnp.float32)
            if i == 0:
                for c in range(Cin):
                    xc = refs[0][:, c, :, :, :].astype(jnp.float32)
                    acc = _conv3d_taps(
                        xc, lambda kt, kh, c=c: w_ref[c, kt, kh],
                        Nb, T, H, W, acc)
            else:
                acc = _conv3d_taps(
                    x, lambda kt, kh: w_ref[kt, kh], Nb, T, H, W * Cin, acc)
            x = (acc + b_ref[...]).reshape(Nb, T // 2, H, W * Cout)
            x = _maxpool_hw(x, H, W, Cout, p)
        Tf, Hf = x.shape[1], x.shape[2]
        Wf = x.shape[3] // C_last
        acc = x[:, :, 0, :]
        for h in range(1, Hf):
            acc = acc + x[:, :, h, :]
        m = acc[..., :C_last]
        for w in range(1, Wf):
            m = m + acc[..., w * C_last:(w + 1) * C_last]
        tfv = m * (1.0 / float(Hf * Wf))
        tfv_ref[...] = tfv.astype(tfv_ref.dtype)
        a = a_ref[...].astype(jnp.float32)
        fused = jnp.maximum(jnp.max(a, axis=1), jnp.max(tfv, axis=1))
        z = jnp.dot(fused, fw_ref[...],
                    preferred_element_type=jnp.float32) + fb_ref[...]
        frame = 1.0 / (1.0 + jnp.exp(-z))
        frame_ref[...] = frame[:, None, :].astype(frame_ref.dtype)
    return body


def kernel(feature, video_feature, a0_w1, a0_w2, a0_s1, a0_b1, a0_s2, a0_b2,
           a1_w1, a1_w2, a1_s1, a1_b1, a1_s2, a1_b2,
           a2_w1, a2_w2, a2_s1, a2_b1, a2_s2, a2_b2,
           a3_w1, a3_w2, a3_s1, a3_b1, a3_s2, a3_b2,
           a4_w1, a4_w2, a4_s1, a4_b1, a4_s2, a4_b2,
           a5_w1, a5_w2, a5_s1, a5_b1, a5_s2, a5_b2,
           v0_w, v0_b, v1_w, v1_b, v2_w, v2_b,
           v3_w, v3_b, v4_w, v4_b, v5_w, v5_b, fc_w, fc_b):
    ablocks = [(a0_w1, a0_w2, a0_s1, a0_b1, a0_s2, a0_b2),
               (a1_w1, a1_w2, a1_s1, a1_b1, a1_s2, a1_b2),
               (a2_w1, a2_w2, a2_s1, a2_b1, a2_s2, a2_b2),
               (a3_w1, a3_w2, a3_s1, a3_b1, a3_s2, a3_b2),
               (a4_w1, a4_w2, a4_s1, a4_b1, a4_s2, a4_b2),
               (a5_w1, a5_w2, a5_s1, a5_b1, a5_s2, a5_b2)]
    vconvs = [(v0_w, v0_b), (v1_w, v1_b), (v2_w, v2_b),
              (v3_w, v3_b), (v4_w, v4_b), (v5_w, v5_b)]

    N, Ta, Fa = feature.shape
    _, Cv, Tv, Hv, Wv = video_feature.shape
    nclass = fc_w.shape[0]

    Nba = 8 if N % 8 == 0 else 1
    adims, ops, specs = [], [feature], [
        pl.BlockSpec((Nba, Ta, Fa), lambda n: (n, 0, 0))]
    H, W, Cin = Ta, Fa, 1
    pc_prev = None
    for i, (w1, w2, s1, b1, s2, b2) in enumerate(ablocks):
        Cout = w1.shape[3]
        w1b = _band2d(w1, W)
        if pc_prev is not None:
            w1b = jnp.matmul(pc_prev, w1b)
        adims.append((H, w1b.shape[1], W * Cout))
        ops += [w1b.astype(jnp.bfloat16),
                _band2d(w2, W).astype(jnp.bfloat16),
                jnp.tile(s1, W)[None, :], jnp.tile(b1, W)[None, :],
                jnp.tile(s2, W)[None, :], jnp.tile(b2, W)[None, :]]
        specs += [pl.BlockSpec(ops[-6].shape, lambda n: (0, 0, 0)),
                  pl.BlockSpec(ops[-5].shape, lambda n: (0, 0, 0)),
                  pl.BlockSpec(ops[-4].shape, lambda n: (0, 0)),
                  pl.BlockSpec(ops[-3].shape, lambda n: (0, 0)),
                  pl.BlockSpec(ops[-2].shape, lambda n: (0, 0)),
                  pl.BlockSpec(ops[-1].shape, lambda n: (0, 0))]
        if i < len(ablocks) - 1:
            pc_prev = _pool_cols(W, Cout)
            H, W = H // 2, W // 2
        Cin = Cout
    C = Cin
    tf_a = pl.pallas_call(
        _make_audio_body(Nba, adims),
        out_shape=jax.ShapeDtypeStruct((N, H, C), feature.dtype),
        grid_spec=pltpu.PrefetchScalarGridSpec(
            num_scalar_prefetch=0, grid=(N // Nba,), in_specs=specs,
            out_specs=pl.BlockSpec((Nba, H, C), lambda n: (n, 0, 0))),
        compiler_params=pltpu.CompilerParams(
            dimension_semantics=("parallel",),
            vmem_limit_bytes=_VMEM_LIMIT),
    )(*ops)

    Nbv = 4 if N % 4 == 0 else 1
    vdims, vops = [], [video_feature]
    vspecs = [pl.BlockSpec((Nbv, Cv, Tv, Hv, Wv), lambda n: (n, 0, 0, 0, 0))]
    pool_pads = (1, 1, 0, 1, 1, 1)
    T, Hh, Ww, Cin = Tv, Hv, Wv, Cv
    for li, (p, (w, b)) in enumerate(zip(pool_pads, vconvs)):
        Cout = w.shape[4]
        vdims.append((T, Hh, Ww, Cin, Cout, p))
        if li == 0:
            wb = jnp.stack([_band3d(w[:, :, :, c:c + 1, :], Ww)
                            for c in range(Cin)])
            vops += [wb.astype(jnp.bfloat16), jnp.tile(b, Ww)[None, :]]
            vspecs += [pl.BlockSpec(vops[-2].shape,
                                    lambda n: (0, 0, 0, 0, 0)),
                       pl.BlockSpec(vops[-1].shape, lambda n: (0, 0))]
        else:
            vops += [_band3d(w, Ww).astype(jnp.bfloat16),
                     jnp.tile(b, Ww)[None, :]]
            vspecs += [pl.BlockSpec(vops[-2].shape, lambda n: (0, 0, 0, 0)),
                       pl.BlockSpec(vops[-1].shape, lambda n: (0, 0))]
        T = T // 2
        Hh = (Hh + 2 * p - 2) // 2 + 1
        Ww = (Ww + 2 * p - 2) // 2 + 1
        Cin = Cout
    vops += [tf_a, fc_w.T, fc_b[None, :]]
    vspecs += [pl.BlockSpec((Nbv,) + tf_a.shape[1:], lambda n: (n, 0, 0)),
               pl.BlockSpec(fc_w.T.shape, lambda n: (0, 0)),
               pl.BlockSpec((1, nclass), lambda n: (0, 0))]
    tf_v, frame3 = pl.pallas_call(
        _make_video_body(Nbv, vdims, C, nclass),
        out_shape=[jax.ShapeDtypeStruct((N, T, C), feature.dtype),
                   jax.ShapeDtypeStruct((N, 1, nclass), feature.dtype)],
        grid_spec=pltpu.PrefetchScalarGridSpec(
            num_scalar_prefetch=0, grid=(N // Nbv,), in_specs=vspecs,
            out_specs=[pl.BlockSpec((Nbv, T, C), lambda n: (n, 0, 0)),
                       pl.BlockSpec((Nbv, 1, nclass), lambda n: (n, 0, 0))]),
        compiler_params=pltpu.CompilerParams(
            dimension_semantics=("parallel",),
            vmem_limit_bytes=_VMEM_LIMIT),
    )(*vops)

    frame = frame3[:, 0, :]
    framewise = jnp.repeat(frame[:, None, :], _UPSAMPLE, axis=1)
    return {"framewise_output": framewise, "clipwise_output": frame,
            "tf_maps_a": tf_a, "tf_maps_v": tf_v}
